# Initial kernel scaffold; baseline (speedup 1.0000x reference)
import jax
import jax.numpy as jnp
from jax.experimental import pallas as pl


def kernel(x, edge_index, Wl0, Wr0, b0, Wl1, Wr1, b1):
    raise NotImplementedError("write your pallas kernel here")


# speedup vs baseline: 4.6497x; 4.6497x over previous
import functools

import jax
import jax.numpy as jnp
from jax import lax
from jax.experimental import pallas as pl
from jax.experimental.pallas import tpu as pltpu
from jax.experimental.pallas import tpu_sc as plsc

NC = 2
NS = 16
NW = NC * NS

CHUNK = 80

_MESH = plsc.VectorSubcoreMesh(core_axis_name="c", subcore_axis_name="s")


def _make_sc_aggregate(n_pad, d, e):
    ew = e // NW
    nch = ew // CHUNK
    rpt = n_pad // NS
    npiece = rpt // CHUNK

    def body(table_h, src_h, dst_h, zf_h, agg_h, acc_sh, sidx, didx, rows):
        c = lax.axis_index("c")
        s = lax.axis_index("s")
        w = c * NS + s
        row0 = s * rpt

        pltpu.sync_copy(zf_h, rows)

        @pl.loop(0, npiece)
        def _(j):
            pltpu.sync_copy(rows, acc_sh.at[pl.ds(row0 + j * CHUNK, CHUNK)])

        plsc.subcore_barrier()

        @pl.loop(0, nch)
        def _(k):
            off = pl.multiple_of(w * ew + k * CHUNK, 8)
            pltpu.sync_copy(src_h.at[pl.ds(off, CHUNK)], sidx)
            pltpu.sync_copy(dst_h.at[pl.ds(off, CHUNK)], didx)
            pltpu.sync_copy(table_h.at[sidx], rows)
            pltpu.sync_copy(rows, acc_sh.at[didx], add=True)

        plsc.subcore_barrier()

        @pl.loop(0, npiece)
        def _(j):
            r = row0 + j * CHUNK
            pltpu.sync_copy(acc_sh.at[pl.ds(r, CHUNK)], rows)
            pltpu.sync_copy(rows, agg_h.at[c, pl.ds(r, CHUNK)])

    return pl.kernel(
        body,
        out_type=jax.ShapeDtypeStruct((NC, n_pad, d), jnp.float32),
        mesh=_MESH,
        scratch_types=[
            pltpu.VMEM_SHARED((n_pad, d), jnp.float32),
            pltpu.VMEM((CHUNK,), jnp.int32),
            pltpu.VMEM((CHUNK,), jnp.int32),
            pltpu.VMEM((CHUNK, d), jnp.float32),
        ],
    )


def _make_sc_count(n_pad, d, e):
    ew = e // NW
    nch = ew // CHUNK
    rpt = n_pad // NS
    npiece = rpt // CHUNK

    def body(dst_h, zf_h, on_h, cnt_h, cnt_sh, didx, rows, ones_v):
        c = lax.axis_index("c")
        s = lax.axis_index("s")
        w = c * NS + s
        row0 = s * rpt

        pltpu.sync_copy(zf_h, rows)
        pltpu.sync_copy(on_h, ones_v)

        @pl.loop(0, npiece)
        def _(j):
            pltpu.sync_copy(rows, cnt_sh.at[pl.ds(row0 + j * CHUNK, CHUNK)])

        plsc.subcore_barrier()

        @pl.loop(0, nch)
        def _(k):
            off = pl.multiple_of(w * ew + k * CHUNK, 8)
            pltpu.sync_copy(dst_h.at[pl.ds(off, CHUNK)], didx)
            pltpu.sync_copy(ones_v, cnt_sh.at[didx], add=True)

        plsc.subcore_barrier()

        @pl.loop(0, npiece)
        def _(j):
            r = row0 + j * CHUNK
            pltpu.sync_copy(cnt_sh.at[pl.ds(r, CHUNK)], rows)
            pltpu.sync_copy(rows, cnt_h.at[c, pl.ds(r, CHUNK)])

    return pl.kernel(
        body,
        out_type=jax.ShapeDtypeStruct((NC, n_pad, d), jnp.float32),
        mesh=_MESH,
        scratch_types=[
            pltpu.VMEM_SHARED((n_pad, d), jnp.float32),
            pltpu.VMEM((CHUNK,), jnp.int32),
            pltpu.VMEM((CHUNK, d), jnp.float32),
            pltpu.VMEM((CHUNK, d), jnp.float32),
        ],
    )


def _combine_body(agg_ref, cnt_ref, h_ref, wl_ref, wr_ref, b_ref, out_ref, *,
                  relu):
    agg = agg_ref[0] + agg_ref[1]
    cnt = cnt_ref[0, :, 0:1] + cnt_ref[1, :, 0:1]
    mean = agg / jnp.maximum(cnt, 1.0)
    acc = jnp.dot(mean, wl_ref[...], preferred_element_type=jnp.float32,
                  precision=lax.Precision.HIGHEST)
    acc += jnp.dot(h_ref[...], wr_ref[...], preferred_element_type=jnp.float32,
                   precision=lax.Precision.HIGHEST)
    acc += b_ref[...]
    out_ref[...] = jnp.maximum(acc, 0.0) if relu else acc


def _combine(agg, cnt, h, wl, wr, b, relu, block):
    n, d = h.shape
    grid = (n // block,)
    return pl.pallas_call(
        functools.partial(_combine_body, relu=relu),
        grid=grid,
        in_specs=[
            pl.BlockSpec((NC, block, d), lambda i: (0, i, 0)),
            pl.BlockSpec((NC, block, d), lambda i: (0, i, 0)),
            pl.BlockSpec((block, d), lambda i: (i, 0)),
            pl.BlockSpec((d, d), lambda i: (0, 0)),
            pl.BlockSpec((d, d), lambda i: (0, 0)),
            pl.BlockSpec((1, d), lambda i: (0, 0)),
        ],
        out_specs=pl.BlockSpec((block, d), lambda i: (i, 0)),
        out_shape=jax.ShapeDtypeStruct((n, d), jnp.float32),
    )(agg, cnt, h, wl, wr, b.reshape(1, d))


@jax.jit
def kernel(x, edge_index, Wl0, Wr0, b0, Wl1, Wr1, b1):
    n, d = x.shape
    e = edge_index.shape[1]
    blk = NS * CHUNK
    n_pad = ((n + blk - 1) // blk) * blk

    src = edge_index[0]
    dst = edge_index[1]
    zeros_feat = jnp.zeros((CHUNK, d), jnp.float32)
    ones_rows = jnp.ones((CHUNK, d), jnp.float32)

    cnt = _make_sc_count(n_pad, d, e)(dst, zeros_feat, ones_rows)
    agg_x = _make_sc_aggregate(n_pad, d, e)(x, src, dst, zeros_feat)
    h1 = _combine(agg_x, cnt, x, Wl0, Wr0, b0, relu=True, block=400)
    agg_h = _make_sc_aggregate(n_pad, d, e)(h1, src, dst, zeros_feat)
    out = _combine(agg_h, cnt, h1, Wl1, Wr1, b1, relu=False, block=400)
    return out
